# Initial kernel scaffold; baseline (speedup 1.0000x reference)
import jax
import jax.numpy as jnp
from jax.experimental import pallas as pl


def kernel(x, edge_index, edge_weight, batch, params):
    raise NotImplementedError("write your pallas kernel here")


# speedup vs baseline: 3.6566x; 3.6566x over previous
import functools

import jax
import jax.numpy as jnp
from jax import lax
from jax.experimental import pallas as pl
from jax.experimental.pallas import tpu as pltpu
from jax.experimental.pallas import tpu_sc as plsc

N = 10000
E = 160000
G = 32
F = 256
FH = 128
NH = N // 2
CH = 80
NT = 16
EPT = E // NT
NCHK = EPT // CH
ATR = 512
AROW = NH + 16 + ATR - 8
ZPT = AROW // NT
WA = 313
WB = 312
MB = 1000


def _sc_prop_body(table, gidx, dstw, out, gbuf, dbuf, didx, rb0, rb1, zb,
                  acc, sem0, sem1):
    c = lax.axis_index("c")
    s = lax.axis_index("s")

    pltpu.sync_copy(gidx.at[c, s], gbuf)
    pltpu.sync_copy(dstw.at[s], dbuf)

    zv = jnp.zeros((16,), jnp.float32)
    lane = lax.iota(jnp.int32, 16)

    def _zero_row(r, carry):
        for k in range(FH // 16):
            zb[r, pl.ds(k * 16, 16)] = zv
        return carry

    lax.fori_loop(0, zb.shape[0], _zero_row, 0)

    for p in range(2):
        lo = p * NH

        for j in range(ZPT // 115):
            pltpu.sync_copy(zb, acc.at[pl.ds(s * ZPT + j * 115, 115)])

        def _remap_row(r, carry):
            for k in range(CH // 16):
                v = dbuf[r, pl.ds(k * 16, 16)]
                ok = (v >= lo) & (v < lo + NH)
                trash = NH + ((r * 16 + lane + k) & (ATR - 1))
                didx[r, pl.ds(k * 16, 16)] = jnp.where(ok, v - lo, trash)
            return carry

        lax.fori_loop(0, NCHK, _remap_row, 0)
        plsc.subcore_barrier()

        def _pair(jj, carry):
            j0 = 2 * jj
            cp0 = pltpu.async_copy(table.at[gbuf.at[j0]], rb0, sem0)
            cp1 = pltpu.async_copy(table.at[gbuf.at[j0 + 1]], rb1, sem1)
            cp0.wait()
            pltpu.sync_copy(rb0, acc.at[didx.at[j0]], add=True)
            cp1.wait()
            pltpu.sync_copy(rb1, acc.at[didx.at[j0 + 1]], add=True)
            return carry

        lax.fori_loop(0, NCHK // 2, _pair, 0)
        cpt = pltpu.async_copy(table.at[gbuf.at[NCHK - 1]], rb0, sem0)
        cpt.wait()
        pltpu.sync_copy(rb0, acc.at[didx.at[NCHK - 1]], add=True)
        plsc.subcore_barrier()

        @pl.when(s < 8)
        def _():
            pltpu.sync_copy(acc.at[pl.ds(s * WA, WA)],
                            out.at[pl.ds(lo + s * WA, WA), c])

        @pl.when(s >= 8)
        def _():
            off = 8 * WA + (s - 8) * WB
            pltpu.sync_copy(acc.at[pl.ds(off, WB)],
                            out.at[pl.ds(lo + off, WB), c])

        plsc.subcore_barrier()


def _make_prop():
    mesh = plsc.VectorSubcoreMesh(core_axis_name="c", subcore_axis_name="s")
    return functools.partial(
        pl.kernel,
        mesh=mesh,
        out_type=jax.ShapeDtypeStruct((N, 2, FH), jnp.float32),
        scratch_types=[
            pltpu.VMEM((NCHK, CH), jnp.int32),
            pltpu.VMEM((NCHK, CH), jnp.int32),
            pltpu.VMEM((NCHK, CH), jnp.int32),
            pltpu.VMEM((CH, FH), jnp.float32),
            pltpu.VMEM((CH, FH), jnp.float32),
            pltpu.VMEM((115, FH), jnp.float32),
            pltpu.VMEM_SHARED((AROW, FH), jnp.float32),
            pltpu.SemaphoreType.DMA,
            pltpu.SemaphoreType.DMA,
        ],
    )(_sc_prop_body)


_prop_cache = []


def _prop_call(table2n, gidx, dstw):
    if not _prop_cache:
        _prop_cache.append(_make_prop())
    return _prop_cache[0](table2n, gidx, dstw)


def _prop(table2n, gidx, dstw):
    return _prop_call(table2n, gidx, dstw).reshape(N, F)


_GRID = N // MB


def _dis_of(d8blk):
    return d8blk[:, :1]


def _disprep_body(degf, dis8):
    deg = degf[:, 0, :1]
    dis8[...] = jnp.broadcast_to(
        jnp.where(deg > 0.0, lax.rsqrt(jnp.maximum(deg, 1e-30)), 0.0), (MB, 8))


def _disprep(degf):
    return pl.pallas_call(
        _disprep_body,
        grid=(_GRID,),
        in_specs=[pl.BlockSpec((MB, 2, FH), lambda i: (i, 0, 0))],
        out_specs=pl.BlockSpec((MB, 8), lambda i: (i, 0)),
        out_shape=jax.ShapeDtypeStruct((N, 8), jnp.float32),
    )(degf)


def _mm_scale_body(h, w, d8, o):
    o[...] = _dis_of(d8[...]) * jnp.dot(h[...], w[...],
                                        preferred_element_type=jnp.float32)


def _mm_scale(h, w, dis8):
    k, oc = w.shape
    return pl.pallas_call(
        _mm_scale_body,
        grid=(_GRID,),
        in_specs=[
            pl.BlockSpec((MB, k), lambda i: (i, 0)),
            pl.BlockSpec((k, oc), lambda i: (0, 0)),
            pl.BlockSpec((MB, 8), lambda i: (i, 0)),
        ],
        out_specs=pl.BlockSpec((MB, oc), lambda i: (i, 0)),
        out_shape=jax.ShapeDtypeStruct((N, oc), jnp.float32),
    )(h, w, dis8)


def _stats_update(i, p, sm, sq):
    @pl.when(i == 0)
    def _():
        sm[...] = jnp.zeros_like(sm)
        sq[...] = jnp.zeros_like(sq)

    sm[...] += jnp.sum(p, axis=0, keepdims=True)
    sq[...] += jnp.sum(p * p, axis=0, keepdims=True)


def _comb1_body(h, w, s, d8, b, pre, sm, sq):
    p = (jnp.dot(h[...], w[...], preferred_element_type=jnp.float32)
         - _dis_of(d8[...]) * s[...] + b[...])
    pre[...] = p
    _stats_update(pl.program_id(0), p, sm, sq)


def _comb1(h, w, s, dis8, b):
    k, oc = w.shape
    return pl.pallas_call(
        _comb1_body,
        grid=(_GRID,),
        in_specs=[
            pl.BlockSpec((MB, k), lambda i: (i, 0)),
            pl.BlockSpec((k, oc), lambda i: (0, 0)),
            pl.BlockSpec((MB, oc), lambda i: (i, 0)),
            pl.BlockSpec((MB, 8), lambda i: (i, 0)),
            pl.BlockSpec((1, oc), lambda i: (0, 0)),
        ],
        out_specs=[
            pl.BlockSpec((MB, oc), lambda i: (i, 0)),
            pl.BlockSpec((1, oc), lambda i: (0, 0)),
            pl.BlockSpec((1, oc), lambda i: (0, 0)),
        ],
        out_shape=[
            jax.ShapeDtypeStruct((N, oc), jnp.float32),
            jax.ShapeDtypeStruct((1, oc), jnp.float32),
            jax.ShapeDtypeStruct((1, oc), jnp.float32),
        ],
    )(h, w, s, dis8, b)


def _comb2_body(h, w0, w2, sa, sc, d8, b, pre, sm, sq):
    d = _dis_of(d8[...])
    p = (jnp.dot(h[...], w0[...] - w2[...], preferred_element_type=jnp.float32)
         - d * sa[...] + 2.0 * d * sc[...] + b[...])
    pre[...] = p
    _stats_update(pl.program_id(0), p, sm, sq)


def _comb2(h, w0, w2, sa, sc, dis8, b):
    k, oc = w0.shape
    return pl.pallas_call(
        _comb2_body,
        grid=(_GRID,),
        in_specs=[
            pl.BlockSpec((MB, k), lambda i: (i, 0)),
            pl.BlockSpec((k, oc), lambda i: (0, 0)),
            pl.BlockSpec((k, oc), lambda i: (0, 0)),
            pl.BlockSpec((MB, oc), lambda i: (i, 0)),
            pl.BlockSpec((MB, oc), lambda i: (i, 0)),
            pl.BlockSpec((MB, 8), lambda i: (i, 0)),
            pl.BlockSpec((1, oc), lambda i: (0, 0)),
        ],
        out_specs=[
            pl.BlockSpec((MB, oc), lambda i: (i, 0)),
            pl.BlockSpec((1, oc), lambda i: (0, 0)),
            pl.BlockSpec((1, oc), lambda i: (0, 0)),
        ],
        out_shape=[
            jax.ShapeDtypeStruct((N, oc), jnp.float32),
            jax.ShapeDtypeStruct((1, oc), jnp.float32),
            jax.ShapeDtypeStruct((1, oc), jnp.float32),
        ],
    )(h, w0, w2, sa, sc, dis8, b)


def _mm3_body(h, v1, s2, w0, w1, w2, d8, b, pre, sm, sq):
    d = _dis_of(d8[...])
    p = (jnp.dot(h[...], w0[...] - w2[...], preferred_element_type=jnp.float32)
         - jnp.dot(v1[...], w1[...], preferred_element_type=jnp.float32)
         - 2.0 * jnp.dot(d * s2[...], w2[...],
                         preferred_element_type=jnp.float32)
         + b[...])
    pre[...] = p
    _stats_update(pl.program_id(0), p, sm, sq)


def _mm3(h, v1, s2, w0, w1, w2, dis8, b):
    k, oc = w0.shape
    return pl.pallas_call(
        _mm3_body,
        grid=(_GRID,),
        in_specs=[
            pl.BlockSpec((MB, k), lambda i: (i, 0)),
            pl.BlockSpec((MB, k), lambda i: (i, 0)),
            pl.BlockSpec((MB, k), lambda i: (i, 0)),
            pl.BlockSpec((k, oc), lambda i: (0, 0)),
            pl.BlockSpec((k, oc), lambda i: (0, 0)),
            pl.BlockSpec((k, oc), lambda i: (0, 0)),
            pl.BlockSpec((MB, 8), lambda i: (i, 0)),
            pl.BlockSpec((1, oc), lambda i: (0, 0)),
        ],
        out_specs=[
            pl.BlockSpec((MB, oc), lambda i: (i, 0)),
            pl.BlockSpec((1, oc), lambda i: (0, 0)),
            pl.BlockSpec((1, oc), lambda i: (0, 0)),
        ],
        out_shape=[
            jax.ShapeDtypeStruct((N, oc), jnp.float32),
            jax.ShapeDtypeStruct((1, oc), jnp.float32),
            jax.ShapeDtypeStruct((1, oc), jnp.float32),
        ],
    )(h, v1, s2, w0, w1, w2, dis8, b)


def _normact_body(want_u, pre, sm, sq, bw, bb, bms, d8, h, *u):
    mean = sm[...] * (1.0 / N)
    m2 = mean * bms[...]
    var = sq[...] * (1.0 / N) - 2.0 * m2 * mean + m2 * m2
    y = bw[...] * (pre[...] - m2) * lax.rsqrt(var + 1e-5) + bb[...]
    hv = jnp.where(y > 0.0, y, 0.2 * y)
    h[...] = hv
    if want_u:
        u[0][...] = _dis_of(d8[...]) * hv


def _normact(pre, sm, sq, bn, dis8, want_u):
    oc = pre.shape[1]
    outs = [jax.ShapeDtypeStruct((N, oc), jnp.float32)]
    ospecs = [pl.BlockSpec((MB, oc), lambda i: (i, 0))]
    if want_u:
        outs.append(jax.ShapeDtypeStruct((N, oc), jnp.float32))
        ospecs.append(pl.BlockSpec((MB, oc), lambda i: (i, 0)))
    res = pl.pallas_call(
        functools.partial(_normact_body, want_u),
        grid=(_GRID,),
        in_specs=[
            pl.BlockSpec((MB, oc), lambda i: (i, 0)),
            pl.BlockSpec((1, oc), lambda i: (0, 0)),
            pl.BlockSpec((1, oc), lambda i: (0, 0)),
            pl.BlockSpec((1, oc), lambda i: (0, 0)),
            pl.BlockSpec((1, oc), lambda i: (0, 0)),
            pl.BlockSpec((1, oc), lambda i: (0, 0)),
            pl.BlockSpec((MB, 8), lambda i: (i, 0)),
        ],
        out_specs=ospecs if want_u else ospecs[0],
        out_shape=outs if want_u else outs[0],
    )(pre, sm, sq, bn['weight'], bn['bias'], bn['mean_scale'], dis8)
    return res


def _uv_body(s, d8, v1, u1):
    d = _dis_of(d8[...])
    v = d * s[...]
    v1[...] = v
    u1[...] = -d * v


def _uv(s, dis8):
    oc = s.shape[1]
    return pl.pallas_call(
        _uv_body,
        grid=(_GRID,),
        in_specs=[
            pl.BlockSpec((MB, oc), lambda i: (i, 0)),
            pl.BlockSpec((MB, 8), lambda i: (i, 0)),
        ],
        out_specs=[
            pl.BlockSpec((MB, oc), lambda i: (i, 0)),
            pl.BlockSpec((MB, oc), lambda i: (i, 0)),
        ],
        out_shape=[
            jax.ShapeDtypeStruct((N, oc), jnp.float32),
            jax.ShapeDtypeStruct((N, oc), jnp.float32),
        ],
    )(s, dis8)


def _ucof_body(s, d8, u):
    d = _dis_of(d8[...])
    u[...] = d * d * s[...]


def _ucof(s, dis8):
    oc = s.shape[1]
    return pl.pallas_call(
        _ucof_body,
        grid=(_GRID,),
        in_specs=[
            pl.BlockSpec((MB, oc), lambda i: (i, 0)),
            pl.BlockSpec((MB, 8), lambda i: (i, 0)),
        ],
        out_specs=pl.BlockSpec((MB, oc), lambda i: (i, 0)),
        out_shape=jax.ShapeDtypeStruct((N, oc), jnp.float32),
    )(s, dis8)


def _res_body(h3, x, s, w0, w1, d8, b, out):
    d = _dis_of(d8[...])
    p = (jnp.dot(h3[...], w0[...], preferred_element_type=jnp.float32)
         - jnp.dot(d * s[...], w1[...], preferred_element_type=jnp.float32)
         + b[...] + x[...])
    out[...] = jnp.maximum(p, 0.0)


def _resconv(h3, x, s, w0, w1, dis8, b):
    k, oc = w0.shape
    return pl.pallas_call(
        _res_body,
        grid=(_GRID,),
        in_specs=[
            pl.BlockSpec((MB, k), lambda i: (i, 0)),
            pl.BlockSpec((MB, oc), lambda i: (i, 0)),
            pl.BlockSpec((MB, k), lambda i: (i, 0)),
            pl.BlockSpec((k, oc), lambda i: (0, 0)),
            pl.BlockSpec((k, oc), lambda i: (0, 0)),
            pl.BlockSpec((MB, 8), lambda i: (i, 0)),
            pl.BlockSpec((1, oc), lambda i: (0, 0)),
        ],
        out_specs=pl.BlockSpec((MB, oc), lambda i: (i, 0)),
        out_shape=jax.ShapeDtypeStruct((N, oc), jnp.float32),
    )(h3, x, s, w0, w1, dis8, b)


def _head_body(h, b3, w1, b1, w2, b2, out, pacc, cacc):
    i = pl.program_id(0)

    @pl.when(i == 0)
    def _():
        pacc[...] = jnp.zeros_like(pacc)
        cacc[...] = jnp.zeros_like(cacc)

    bids = b3[0]
    onehot = (lax.broadcasted_iota(jnp.int32, (G, MB), 0)
              == bids).astype(jnp.float32)
    pacc[...] += jnp.dot(onehot, h[...], preferred_element_type=jnp.float32)
    cacc[...] += jnp.broadcast_to(
        jnp.sum(onehot, axis=1, keepdims=True), (G, 8))

    @pl.when(i == _GRID - 1)
    def _():
        pooled = pacc[...] / jnp.maximum(cacc[:, :1], 1.0)
        o = jnp.tanh(jnp.dot(pooled, w1[...],
                             preferred_element_type=jnp.float32) + b1[...])
        out[...] = jnp.dot(o, w2[...],
                           preferred_element_type=jnp.float32) + b2[...]


def _head(h, batch3, w1, b1, w2, b2):
    return pl.pallas_call(
        _head_body,
        grid=(_GRID,),
        in_specs=[
            pl.BlockSpec((MB, 512), lambda i: (i, 0)),
            pl.BlockSpec((1, 1, MB), lambda i: (i, 0, 0)),
            pl.BlockSpec((512, 64), lambda i: (0, 0)),
            pl.BlockSpec((1, 64), lambda i: (0, 0)),
            pl.BlockSpec((64, 12), lambda i: (0, 0)),
            pl.BlockSpec((1, 12), lambda i: (0, 0)),
        ],
        out_specs=pl.BlockSpec((G, 12), lambda i: (0, 0)),
        out_shape=jax.ShapeDtypeStruct((G, 12), jnp.float32),
        scratch_shapes=[
            pltpu.VMEM((G, 512), jnp.float32),
            pltpu.VMEM((G, 8), jnp.float32),
        ],
    )(h, batch3, w1, b1, w2, b2)


def kernel(x, edge_index, edge_weight, batch, params):
    del edge_weight
    src = edge_index[0]
    dst = edge_index[1]

    gidx = jnp.stack([2 * src, 2 * src + 1]).reshape(2, NT, NCHK, CH)
    dstw = dst.reshape(NT, NCHK, CH)
    srcw = src.reshape(NT, NCHK, CH)

    ones_t = jnp.ones((2 * N, FH), jnp.float32)
    degf = _prop_call(ones_t, gidx, srcw)
    dis8 = _disprep(degf)

    def t2(a):
        return a.reshape(2 * N, FH)

    def b2d(v):
        return v.reshape(1, -1)

    h = x
    for blk in range(4):
        p1 = params['conv%d' % (blk * 4 + 1)]
        p2 = params['conv%d' % (blk * 4 + 2)]
        p3 = params['conv%d' % (blk * 4 + 3)]
        p4 = params['conv%d' % (blk * 4 + 4)]
        bn1 = params['bn%d' % (blk * 4 + 1)]
        bn2 = params['bn%d' % (blk * 4 + 2)]
        bn3 = params['bn%d' % (blk * 4 + 3)]
        bn1 = {k: b2d(v) for k, v in bn1.items()}
        bn2 = {k: b2d(v) for k, v in bn2.items()}
        bn3 = {k: b2d(v) for k, v in bn3.items()}

        y1u = _mm_scale(h, p1['Ws'][1], dis8)
        s1 = _prop(t2(y1u), gidx, dstw)
        pre1, sm, sq = _comb1(h, p1['Ws'][0], s1, dis8, b2d(p1['b']))
        h1, u0 = _normact(pre1, sm, sq, bn1, dis8, True)

        s1b = _prop(t2(u0), gidx, dstw)
        v1, u1 = _uv(s1b, dis8)
        s2 = _prop(t2(u1), gidx, dstw)
        pre2, sm, sq = _mm3(h1, v1, s2, p2['Ws'][0], p2['Ws'][1],
                            p2['Ws'][2], dis8, b2d(p2['b']))
        h2 = _normact(pre2, sm, sq, bn2, dis8, False)

        y1u = _mm_scale(h2, p3['Ws'][1], dis8)
        y2u = _mm_scale(h2, p3['Ws'][2], dis8)
        sa = _prop(t2(y1u), gidx, dstw)
        sb = _prop(t2(y2u), gidx, dstw)
        uc = _ucof(sb, dis8)
        sc = _prop(t2(uc), gidx, dstw)
        pre3, sm, sq = _comb2(h2, p3['Ws'][0], p3['Ws'][2], sa, sc,
                              dis8, b2d(p3['b']))
        h3, u3 = _normact(pre3, sm, sq, bn3, dis8, True)

        s4 = _prop(t2(u3), gidx, dstw)
        h = _resconv(h3, x, s4, p4['Ws'][0], p4['Ws'][1], dis8, b2d(p4['b']))

    batch3 = batch.reshape(_GRID, 1, MB)
    return _head(h, batch3, params['linear1']['W'], b2d(params['linear1']['b']),
                 params['linear2']['W'], b2d(params['linear2']['b']))


# speedup vs baseline: 4.0605x; 1.1105x over previous
import functools

import jax
import jax.numpy as jnp
from jax import lax
from jax.experimental import pallas as pl
from jax.experimental.pallas import tpu as pltpu
from jax.experimental.pallas import tpu_sc as plsc

N = 10000
E = 160000
G = 32
F = 256
FH = 128
NH = N // 2
CH = 80
NT = 16
EPT = E // NT
WIN = EPT + 3 * CH
NWCH = WIN // CH
ATR = 512
AROW = NH + 16 + ATR - 8
ZPT = AROW // NT
WA = 313
WB = 312
MB = 1000


def _sc_prop_body(table, gidx, dstw, meta, out, gbuf, dbuf, didx, mbuf,
                  rb0, rb1, zb, acc, sem0, sem1):
    c = lax.axis_index("c")
    s = lax.axis_index("s")

    zv = jnp.zeros((16,), jnp.float32)
    lane = lax.iota(jnp.int32, 16)

    def _zero_row(r, carry):
        for k in range(FH // 16):
            zb[r, pl.ds(k * 16, 16)] = zv
        return carry

    lax.fori_loop(0, zb.shape[0], _zero_row, 0)

    for p in range(2):
        lo = p * NH

        pltpu.sync_copy(meta.at[p, s], mbuf)
        mv = mbuf[0, pl.ds(0, 16)]
        astart = pl.multiple_of(mv[0], 128)
        estart = mv[1]
        eend = mv[2]
        npairs = mv[3]

        pltpu.sync_copy(gidx.at[c, 0, pl.ds(astart, WIN)], gbuf)
        pltpu.sync_copy(dstw.at[pl.ds(astart, WIN)], dbuf)

        for j in range(ZPT // 115):
            pltpu.sync_copy(zb, acc.at[pl.ds(s * ZPT + j * 115, 115)])

        def _remap_row(r, carry):
            for k in range(CH // 16):
                i16 = r * CH + k * 16
                pos = astart + i16 + lane
                ok = (pos >= estart) & (pos < eend)
                d = dbuf[pl.ds(i16, 16)]
                g = gbuf[pl.ds(i16, 16)]
                trash = NH + ((r * 16 + lane + k) & (ATR - 1))
                didx[r, pl.ds(k * 16, 16)] = jnp.where(ok, d - lo, trash)
                gbuf[pl.ds(i16, 16)] = jnp.where(ok, g, lane)
            return carry

        lax.fori_loop(0, NWCH, _remap_row, 0)
        plsc.subcore_barrier()

        def _pair(jj, carry):
            j0 = 2 * jj
            cp0 = pltpu.async_copy(table.at[gbuf.at[pl.ds(j0 * CH, CH)]],
                                   rb0, sem0)
            cp1 = pltpu.async_copy(table.at[gbuf.at[pl.ds(j0 * CH + CH, CH)]],
                                   rb1, sem1)
            cp0.wait()
            pltpu.sync_copy(rb0, acc.at[didx.at[j0]], add=True)
            cp1.wait()
            pltpu.sync_copy(rb1, acc.at[didx.at[j0 + 1]], add=True)
            return carry

        lax.fori_loop(0, npairs, _pair, 0)
        plsc.subcore_barrier()

        @pl.when(s < 8)
        def _():
            pltpu.sync_copy(acc.at[pl.ds(s * WA, WA)],
                            out.at[pl.ds(lo + s * WA, WA), c])

        @pl.when(s >= 8)
        def _():
            off = 8 * WA + (s - 8) * WB
            pltpu.sync_copy(acc.at[pl.ds(off, WB)],
                            out.at[pl.ds(lo + off, WB), c])

        plsc.subcore_barrier()


def _make_prop():
    mesh = plsc.VectorSubcoreMesh(core_axis_name="c", subcore_axis_name="s")
    return functools.partial(
        pl.kernel,
        mesh=mesh,
        out_type=jax.ShapeDtypeStruct((N, 2, FH), jnp.float32),
        scratch_types=[
            pltpu.VMEM((WIN,), jnp.int32),
            pltpu.VMEM((WIN,), jnp.int32),
            pltpu.VMEM((NWCH, CH), jnp.int32),
            pltpu.VMEM((1, 16), jnp.int32),
            pltpu.VMEM((CH, FH), jnp.float32),
            pltpu.VMEM((CH, FH), jnp.float32),
            pltpu.VMEM((115, FH), jnp.float32),
            pltpu.VMEM_SHARED((AROW, FH), jnp.float32),
            pltpu.SemaphoreType.DMA,
            pltpu.SemaphoreType.DMA,
        ],
    )(_sc_prop_body)


_prop_cache = []


def _prop_call(table2n, gidx, dstw, meta):
    if not _prop_cache:
        _prop_cache.append(_make_prop())
    return _prop_cache[0](table2n, gidx, dstw, meta)


def _prop(table2n, gidx, dstw, meta):
    return _prop_call(table2n, gidx, dstw, meta).reshape(N, F)


_GRID = N // MB


def _dis_of(d8blk):
    return d8blk[:, :1]


def _disprep_body(degf, dis8):
    deg = degf[:, 0, :1]
    dis8[...] = jnp.broadcast_to(
        jnp.where(deg > 0.0, lax.rsqrt(jnp.maximum(deg, 1e-30)), 0.0), (MB, 8))


def _disprep(degf):
    return pl.pallas_call(
        _disprep_body,
        grid=(_GRID,),
        in_specs=[pl.BlockSpec((MB, 2, FH), lambda i: (i, 0, 0))],
        out_specs=pl.BlockSpec((MB, 8), lambda i: (i, 0)),
        out_shape=jax.ShapeDtypeStruct((N, 8), jnp.float32),
    )(degf)


def _mm_scale_body(h, w, d8, o):
    o[...] = _dis_of(d8[...]) * jnp.dot(h[...], w[...],
                                        preferred_element_type=jnp.float32)


def _mm_scale(h, w, dis8):
    k, oc = w.shape
    return pl.pallas_call(
        _mm_scale_body,
        grid=(_GRID,),
        in_specs=[
            pl.BlockSpec((MB, k), lambda i: (i, 0)),
            pl.BlockSpec((k, oc), lambda i: (0, 0)),
            pl.BlockSpec((MB, 8), lambda i: (i, 0)),
        ],
        out_specs=pl.BlockSpec((MB, oc), lambda i: (i, 0)),
        out_shape=jax.ShapeDtypeStruct((N, oc), jnp.float32),
    )(h, w, dis8)


def _stats_update(i, p, sm, sq):
    @pl.when(i == 0)
    def _():
        sm[...] = jnp.zeros_like(sm)
        sq[...] = jnp.zeros_like(sq)

    sm[...] += jnp.sum(p, axis=0, keepdims=True)
    sq[...] += jnp.sum(p * p, axis=0, keepdims=True)


def _comb1_body(h, w, s, d8, b, pre, sm, sq):
    p = (jnp.dot(h[...], w[...], preferred_element_type=jnp.float32)
         - _dis_of(d8[...]) * s[...] + b[...])
    pre[...] = p
    _stats_update(pl.program_id(0), p, sm, sq)


def _comb1(h, w, s, dis8, b):
    k, oc = w.shape
    return pl.pallas_call(
        _comb1_body,
        grid=(_GRID,),
        in_specs=[
            pl.BlockSpec((MB, k), lambda i: (i, 0)),
            pl.BlockSpec((k, oc), lambda i: (0, 0)),
            pl.BlockSpec((MB, oc), lambda i: (i, 0)),
            pl.BlockSpec((MB, 8), lambda i: (i, 0)),
            pl.BlockSpec((1, oc), lambda i: (0, 0)),
        ],
        out_specs=[
            pl.BlockSpec((MB, oc), lambda i: (i, 0)),
            pl.BlockSpec((1, oc), lambda i: (0, 0)),
            pl.BlockSpec((1, oc), lambda i: (0, 0)),
        ],
        out_shape=[
            jax.ShapeDtypeStruct((N, oc), jnp.float32),
            jax.ShapeDtypeStruct((1, oc), jnp.float32),
            jax.ShapeDtypeStruct((1, oc), jnp.float32),
        ],
    )(h, w, s, dis8, b)


def _comb2_body(h, w0, w2, sa, sc, d8, b, pre, sm, sq):
    d = _dis_of(d8[...])
    p = (jnp.dot(h[...], w0[...] - w2[...], preferred_element_type=jnp.float32)
         - d * sa[...] + 2.0 * d * sc[...] + b[...])
    pre[...] = p
    _stats_update(pl.program_id(0), p, sm, sq)


def _comb2(h, w0, w2, sa, sc, dis8, b):
    k, oc = w0.shape
    return pl.pallas_call(
        _comb2_body,
        grid=(_GRID,),
        in_specs=[
            pl.BlockSpec((MB, k), lambda i: (i, 0)),
            pl.BlockSpec((k, oc), lambda i: (0, 0)),
            pl.BlockSpec((k, oc), lambda i: (0, 0)),
            pl.BlockSpec((MB, oc), lambda i: (i, 0)),
            pl.BlockSpec((MB, oc), lambda i: (i, 0)),
            pl.BlockSpec((MB, 8), lambda i: (i, 0)),
            pl.BlockSpec((1, oc), lambda i: (0, 0)),
        ],
        out_specs=[
            pl.BlockSpec((MB, oc), lambda i: (i, 0)),
            pl.BlockSpec((1, oc), lambda i: (0, 0)),
            pl.BlockSpec((1, oc), lambda i: (0, 0)),
        ],
        out_shape=[
            jax.ShapeDtypeStruct((N, oc), jnp.float32),
            jax.ShapeDtypeStruct((1, oc), jnp.float32),
            jax.ShapeDtypeStruct((1, oc), jnp.float32),
        ],
    )(h, w0, w2, sa, sc, dis8, b)


def _mm3_body(h, v1, s2, w0, w1, w2, d8, b, pre, sm, sq):
    d = _dis_of(d8[...])
    p = (jnp.dot(h[...], w0[...] - w2[...], preferred_element_type=jnp.float32)
         - jnp.dot(v1[...], w1[...], preferred_element_type=jnp.float32)
         - 2.0 * jnp.dot(d * s2[...], w2[...],
                         preferred_element_type=jnp.float32)
         + b[...])
    pre[...] = p
    _stats_update(pl.program_id(0), p, sm, sq)


def _mm3(h, v1, s2, w0, w1, w2, dis8, b):
    k, oc = w0.shape
    return pl.pallas_call(
        _mm3_body,
        grid=(_GRID,),
        in_specs=[
            pl.BlockSpec((MB, k), lambda i: (i, 0)),
            pl.BlockSpec((MB, k), lambda i: (i, 0)),
            pl.BlockSpec((MB, k), lambda i: (i, 0)),
            pl.BlockSpec((k, oc), lambda i: (0, 0)),
            pl.BlockSpec((k, oc), lambda i: (0, 0)),
            pl.BlockSpec((k, oc), lambda i: (0, 0)),
            pl.BlockSpec((MB, 8), lambda i: (i, 0)),
            pl.BlockSpec((1, oc), lambda i: (0, 0)),
        ],
        out_specs=[
            pl.BlockSpec((MB, oc), lambda i: (i, 0)),
            pl.BlockSpec((1, oc), lambda i: (0, 0)),
            pl.BlockSpec((1, oc), lambda i: (0, 0)),
        ],
        out_shape=[
            jax.ShapeDtypeStruct((N, oc), jnp.float32),
            jax.ShapeDtypeStruct((1, oc), jnp.float32),
            jax.ShapeDtypeStruct((1, oc), jnp.float32),
        ],
    )(h, v1, s2, w0, w1, w2, dis8, b)


def _normact_body(want_u, pre, sm, sq, bw, bb, bms, d8, h, *u):
    mean = sm[...] * (1.0 / N)
    m2 = mean * bms[...]
    var = sq[...] * (1.0 / N) - 2.0 * m2 * mean + m2 * m2
    y = bw[...] * (pre[...] - m2) * lax.rsqrt(var + 1e-5) + bb[...]
    hv = jnp.where(y > 0.0, y, 0.2 * y)
    h[...] = hv
    if want_u:
        u[0][...] = _dis_of(d8[...]) * hv


def _normact(pre, sm, sq, bn, dis8, want_u):
    oc = pre.shape[1]
    outs = [jax.ShapeDtypeStruct((N, oc), jnp.float32)]
    ospecs = [pl.BlockSpec((MB, oc), lambda i: (i, 0))]
    if want_u:
        outs.append(jax.ShapeDtypeStruct((N, oc), jnp.float32))
        ospecs.append(pl.BlockSpec((MB, oc), lambda i: (i, 0)))
    res = pl.pallas_call(
        functools.partial(_normact_body, want_u),
        grid=(_GRID,),
        in_specs=[
            pl.BlockSpec((MB, oc), lambda i: (i, 0)),
            pl.BlockSpec((1, oc), lambda i: (0, 0)),
            pl.BlockSpec((1, oc), lambda i: (0, 0)),
            pl.BlockSpec((1, oc), lambda i: (0, 0)),
            pl.BlockSpec((1, oc), lambda i: (0, 0)),
            pl.BlockSpec((1, oc), lambda i: (0, 0)),
            pl.BlockSpec((MB, 8), lambda i: (i, 0)),
        ],
        out_specs=ospecs if want_u else ospecs[0],
        out_shape=outs if want_u else outs[0],
    )(pre, sm, sq, bn['weight'], bn['bias'], bn['mean_scale'], dis8)
    return res


def _uv_body(s, d8, v1, u1):
    d = _dis_of(d8[...])
    v = d * s[...]
    v1[...] = v
    u1[...] = -d * v


def _uv(s, dis8):
    oc = s.shape[1]
    return pl.pallas_call(
        _uv_body,
        grid=(_GRID,),
        in_specs=[
            pl.BlockSpec((MB, oc), lambda i: (i, 0)),
            pl.BlockSpec((MB, 8), lambda i: (i, 0)),
        ],
        out_specs=[
            pl.BlockSpec((MB, oc), lambda i: (i, 0)),
            pl.BlockSpec((MB, oc), lambda i: (i, 0)),
        ],
        out_shape=[
            jax.ShapeDtypeStruct((N, oc), jnp.float32),
            jax.ShapeDtypeStruct((N, oc), jnp.float32),
        ],
    )(s, dis8)


def _ucof_body(s, d8, u):
    d = _dis_of(d8[...])
    u[...] = d * d * s[...]


def _ucof(s, dis8):
    oc = s.shape[1]
    return pl.pallas_call(
        _ucof_body,
        grid=(_GRID,),
        in_specs=[
            pl.BlockSpec((MB, oc), lambda i: (i, 0)),
            pl.BlockSpec((MB, 8), lambda i: (i, 0)),
        ],
        out_specs=pl.BlockSpec((MB, oc), lambda i: (i, 0)),
        out_shape=jax.ShapeDtypeStruct((N, oc), jnp.float32),
    )(s, dis8)


def _res_body(h3, x, s, w0, w1, d8, b, out):
    d = _dis_of(d8[...])
    p = (jnp.dot(h3[...], w0[...], preferred_element_type=jnp.float32)
         - jnp.dot(d * s[...], w1[...], preferred_element_type=jnp.float32)
         + b[...] + x[...])
    out[...] = jnp.maximum(p, 0.0)


def _resconv(h3, x, s, w0, w1, dis8, b):
    k, oc = w0.shape
    return pl.pallas_call(
        _res_body,
        grid=(_GRID,),
        in_specs=[
            pl.BlockSpec((MB, k), lambda i: (i, 0)),
            pl.BlockSpec((MB, oc), lambda i: (i, 0)),
            pl.BlockSpec((MB, k), lambda i: (i, 0)),
            pl.BlockSpec((k, oc), lambda i: (0, 0)),
            pl.BlockSpec((k, oc), lambda i: (0, 0)),
            pl.BlockSpec((MB, 8), lambda i: (i, 0)),
            pl.BlockSpec((1, oc), lambda i: (0, 0)),
        ],
        out_specs=pl.BlockSpec((MB, oc), lambda i: (i, 0)),
        out_shape=jax.ShapeDtypeStruct((N, oc), jnp.float32),
    )(h3, x, s, w0, w1, dis8, b)


def _head_body(h, b3, w1, b1, w2, b2, out, pacc, cacc):
    i = pl.program_id(0)

    @pl.when(i == 0)
    def _():
        pacc[...] = jnp.zeros_like(pacc)
        cacc[...] = jnp.zeros_like(cacc)

    bids = b3[0]
    onehot = (lax.broadcasted_iota(jnp.int32, (G, MB), 0)
              == bids).astype(jnp.float32)
    pacc[...] += jnp.dot(onehot, h[...], preferred_element_type=jnp.float32)
    cacc[...] += jnp.broadcast_to(
        jnp.sum(onehot, axis=1, keepdims=True), (G, 8))

    @pl.when(i == _GRID - 1)
    def _():
        pooled = pacc[...] / jnp.maximum(cacc[:, :1], 1.0)
        o = jnp.tanh(jnp.dot(pooled, w1[...],
                             preferred_element_type=jnp.float32) + b1[...])
        out[...] = jnp.dot(o, w2[...],
                           preferred_element_type=jnp.float32) + b2[...]


def _head(h, batch3, w1, b1, w2, b2):
    return pl.pallas_call(
        _head_body,
        grid=(_GRID,),
        in_specs=[
            pl.BlockSpec((MB, 512), lambda i: (i, 0)),
            pl.BlockSpec((1, 1, MB), lambda i: (i, 0, 0)),
            pl.BlockSpec((512, 64), lambda i: (0, 0)),
            pl.BlockSpec((1, 64), lambda i: (0, 0)),
            pl.BlockSpec((64, 12), lambda i: (0, 0)),
            pl.BlockSpec((1, 12), lambda i: (0, 0)),
        ],
        out_specs=pl.BlockSpec((G, 12), lambda i: (0, 0)),
        out_shape=jax.ShapeDtypeStruct((G, 12), jnp.float32),
        scratch_shapes=[
            pltpu.VMEM((G, 512), jnp.float32),
            pltpu.VMEM((G, 8), jnp.float32),
        ],
    )(h, batch3, w1, b1, w2, b2)


def kernel(x, edge_index, edge_weight, batch, params):
    del edge_weight
    src = edge_index[0]
    dst = edge_index[1]

    def windows(scatter_ids, gather_ids):
        perm = jnp.argsort((scatter_ids >= NH).astype(jnp.int32))
        sp = scatter_ids[perm]
        gp = gather_ids[perm]
        cnt0 = jnp.sum((scatter_ids < NH).astype(jnp.int32))
        t = jnp.arange(NT, dtype=jnp.int32)
        bounds = jnp.stack([
            (cnt0 * t) // NT, (cnt0 * (t + 1)) // NT,
            cnt0 + ((E - cnt0) * t) // NT,
            cnt0 + ((E - cnt0) * (t + 1)) // NT,
        ]).reshape(2, 2, NT)
        start = bounds[:, 0]
        end = bounds[:, 1]
        astart = jnp.minimum(start - (start % 128), E - WIN)
        npairs = (end - astart + 2 * CH - 1) // (2 * CH)
        meta = jnp.stack(
            [astart, start, end, npairs] + [jnp.zeros_like(start)] * 12,
            axis=-1).astype(jnp.int32).reshape(2, NT, 1, 16)
        gidx = jnp.stack([2 * gp, 2 * gp + 1]).reshape(2, 1, E)
        return gidx, sp, meta

    gidx, dstp, meta = windows(dst, src)
    gidx_d, srcp_d, meta_d = windows(src, src)

    ones_t = jnp.ones((2 * N, FH), jnp.float32)
    degf = _prop_call(ones_t, gidx_d, srcp_d, meta_d)
    dis8 = _disprep(degf)

    def t2(a):
        return a.reshape(2 * N, FH)

    def b2d(v):
        return v.reshape(1, -1)

    h = x
    for blk in range(4):
        p1 = params['conv%d' % (blk * 4 + 1)]
        p2 = params['conv%d' % (blk * 4 + 2)]
        p3 = params['conv%d' % (blk * 4 + 3)]
        p4 = params['conv%d' % (blk * 4 + 4)]
        bn1 = params['bn%d' % (blk * 4 + 1)]
        bn2 = params['bn%d' % (blk * 4 + 2)]
        bn3 = params['bn%d' % (blk * 4 + 3)]
        bn1 = {k: b2d(v) for k, v in bn1.items()}
        bn2 = {k: b2d(v) for k, v in bn2.items()}
        bn3 = {k: b2d(v) for k, v in bn3.items()}

        y1u = _mm_scale(h, p1['Ws'][1], dis8)
        s1 = _prop(t2(y1u), gidx, dstp, meta)
        pre1, sm, sq = _comb1(h, p1['Ws'][0], s1, dis8, b2d(p1['b']))
        h1, u0 = _normact(pre1, sm, sq, bn1, dis8, True)

        s1b = _prop(t2(u0), gidx, dstp, meta)
        v1, u1 = _uv(s1b, dis8)
        s2 = _prop(t2(u1), gidx, dstp, meta)
        pre2, sm, sq = _mm3(h1, v1, s2, p2['Ws'][0], p2['Ws'][1],
                            p2['Ws'][2], dis8, b2d(p2['b']))
        h2 = _normact(pre2, sm, sq, bn2, dis8, False)

        y1u = _mm_scale(h2, p3['Ws'][1], dis8)
        y2u = _mm_scale(h2, p3['Ws'][2], dis8)
        sa = _prop(t2(y1u), gidx, dstp, meta)
        sb = _prop(t2(y2u), gidx, dstp, meta)
        uc = _ucof(sb, dis8)
        sc = _prop(t2(uc), gidx, dstp, meta)
        pre3, sm, sq = _comb2(h2, p3['Ws'][0], p3['Ws'][2], sa, sc,
                              dis8, b2d(p3['b']))
        h3, u3 = _normact(pre3, sm, sq, bn3, dis8, True)

        s4 = _prop(t2(u3), gidx, dstp, meta)
        h = _resconv(h3, x, s4, p4['Ws'][0], p4['Ws'][1], dis8, b2d(p4['b']))

    batch3 = batch.reshape(_GRID, 1, MB)
    return _head(h, batch3, params['linear1']['W'], b2d(params['linear1']['b']),
                 params['linear2']['W'], b2d(params['linear2']['b']))


# speedup vs baseline: 4.3037x; 1.0599x over previous
import functools

import jax
import jax.numpy as jnp
from jax import lax
from jax.experimental import pallas as pl
from jax.experimental.pallas import tpu as pltpu
from jax.experimental.pallas import tpu_sc as plsc

N = 10000
E = 160000
G = 32
F = 256
FH = 128
NH = N // 2
CH = 128
NT = 16
EPT = E // NT
WIN = EPT + 240
NWCH = WIN // CH
ATR = 512
AROW = NH + 16 + ATR - 8
ZPT = AROW // NT
WA = 313
WB = 312
MB = 1000


def _sc_prop_body(table, gidx, dstw, meta, out, gbuf, dbuf, didx, mbuf,
                  rb0, rb1, zb, acc, sem0, sem1):
    c = lax.axis_index("c")
    s = lax.axis_index("s")

    zv = jnp.zeros((16,), jnp.float32)
    lane = lax.iota(jnp.int32, 16)

    def _zero_row(r, carry):
        for k in range(FH // 16):
            zb[r, pl.ds(k * 16, 16)] = zv
        return carry

    lax.fori_loop(0, zb.shape[0], _zero_row, 0)

    for p in range(2):
        lo = p * NH

        pltpu.sync_copy(meta.at[p, s], mbuf)
        mv = mbuf[0, pl.ds(0, 16)]
        astart = pl.multiple_of(mv[0], 128)
        estart = mv[1]
        eend = mv[2]
        npairs = mv[3]

        pltpu.sync_copy(gidx.at[c, 0, pl.ds(astart, WIN)], gbuf)
        pltpu.sync_copy(dstw.at[pl.ds(astart, WIN)], dbuf)

        for j in range(ZPT // 115):
            pltpu.sync_copy(zb, acc.at[pl.ds(s * ZPT + j * 115, 115)])

        def _remap_row(r, carry):
            for k in range(CH // 16):
                i16 = r * CH + k * 16
                pos = astart + i16 + lane
                ok = (pos >= estart) & (pos < eend)
                d = dbuf[pl.ds(i16, 16)]
                g = gbuf[pl.ds(i16, 16)]
                trash = NH + ((r * 16 + lane + k) & (ATR - 1))
                didx[r, pl.ds(k * 16, 16)] = jnp.where(ok, d - lo, trash)
                gbuf[pl.ds(i16, 16)] = jnp.where(ok, g, lane)
            return carry

        lax.fori_loop(0, NWCH, _remap_row, 0)
        plsc.subcore_barrier()

        def _pair(jj, carry):
            j0 = 2 * jj
            cp0 = pltpu.async_copy(table.at[gbuf.at[pl.ds(j0 * CH, CH)]],
                                   rb0, sem0)
            cp1 = pltpu.async_copy(table.at[gbuf.at[pl.ds(j0 * CH + CH, CH)]],
                                   rb1, sem1)
            cp0.wait()
            pltpu.sync_copy(rb0, acc.at[didx.at[j0]], add=True)
            cp1.wait()
            pltpu.sync_copy(rb1, acc.at[didx.at[j0 + 1]], add=True)
            return carry

        lax.fori_loop(0, npairs, _pair, 0)
        plsc.subcore_barrier()

        @pl.when(s < 8)
        def _():
            pltpu.sync_copy(acc.at[pl.ds(s * WA, WA)],
                            out.at[pl.ds(lo + s * WA, WA), c])

        @pl.when(s >= 8)
        def _():
            off = 8 * WA + (s - 8) * WB
            pltpu.sync_copy(acc.at[pl.ds(off, WB)],
                            out.at[pl.ds(lo + off, WB), c])

        plsc.subcore_barrier()


def _make_prop():
    mesh = plsc.VectorSubcoreMesh(core_axis_name="c", subcore_axis_name="s")
    return functools.partial(
        pl.kernel,
        mesh=mesh,
        out_type=jax.ShapeDtypeStruct((N, 2, FH), jnp.float32),
        scratch_types=[
            pltpu.VMEM((WIN,), jnp.int32),
            pltpu.VMEM((WIN,), jnp.int32),
            pltpu.VMEM((NWCH, CH), jnp.int32),
            pltpu.VMEM((1, 16), jnp.int32),
            pltpu.VMEM((CH, FH), jnp.float32),
            pltpu.VMEM((CH, FH), jnp.float32),
            pltpu.VMEM((115, FH), jnp.float32),
            pltpu.VMEM_SHARED((AROW, FH), jnp.float32),
            pltpu.SemaphoreType.DMA,
            pltpu.SemaphoreType.DMA,
        ],
    )(_sc_prop_body)


_prop_cache = []


def _prop_call(table2n, gidx, dstw, meta):
    if not _prop_cache:
        _prop_cache.append(_make_prop())
    return _prop_cache[0](table2n, gidx, dstw, meta)


def _prop(table2n, gidx, dstw, meta):
    return _prop_call(table2n, gidx, dstw, meta).reshape(N, F)


_GRID = N // MB


def _dis_of(d8blk):
    return d8blk[:, :1]


def _disprep_body(degf, dis8):
    deg = degf[:, 0, :1]
    dis8[...] = jnp.broadcast_to(
        jnp.where(deg > 0.0, lax.rsqrt(jnp.maximum(deg, 1e-30)), 0.0), (MB, 8))


def _disprep(degf):
    return pl.pallas_call(
        _disprep_body,
        grid=(_GRID,),
        in_specs=[pl.BlockSpec((MB, 2, FH), lambda i: (i, 0, 0))],
        out_specs=pl.BlockSpec((MB, 8), lambda i: (i, 0)),
        out_shape=jax.ShapeDtypeStruct((N, 8), jnp.float32),
    )(degf)


def _mm_scale_body(h, w, d8, o):
    o[...] = _dis_of(d8[...]) * jnp.dot(h[...], w[...],
                                        preferred_element_type=jnp.float32)


def _mm_scale(h, w, dis8):
    k, oc = w.shape
    return pl.pallas_call(
        _mm_scale_body,
        grid=(_GRID,),
        in_specs=[
            pl.BlockSpec((MB, k), lambda i: (i, 0)),
            pl.BlockSpec((k, oc), lambda i: (0, 0)),
            pl.BlockSpec((MB, 8), lambda i: (i, 0)),
        ],
        out_specs=pl.BlockSpec((MB, oc), lambda i: (i, 0)),
        out_shape=jax.ShapeDtypeStruct((N, oc), jnp.float32),
    )(h, w, dis8)


def _stats_update(i, p, sm, sq):
    @pl.when(i == 0)
    def _():
        sm[...] = jnp.zeros_like(sm)
        sq[...] = jnp.zeros_like(sq)

    sm[...] += jnp.sum(p, axis=0, keepdims=True)
    sq[...] += jnp.sum(p * p, axis=0, keepdims=True)


def _comb1_body(h, w, s, d8, b, pre, sm, sq):
    p = (jnp.dot(h[...], w[...], preferred_element_type=jnp.float32)
         - _dis_of(d8[...]) * s[...] + b[...])
    pre[...] = p
    _stats_update(pl.program_id(0), p, sm, sq)


def _comb1(h, w, s, dis8, b):
    k, oc = w.shape
    return pl.pallas_call(
        _comb1_body,
        grid=(_GRID,),
        in_specs=[
            pl.BlockSpec((MB, k), lambda i: (i, 0)),
            pl.BlockSpec((k, oc), lambda i: (0, 0)),
            pl.BlockSpec((MB, oc), lambda i: (i, 0)),
            pl.BlockSpec((MB, 8), lambda i: (i, 0)),
            pl.BlockSpec((1, oc), lambda i: (0, 0)),
        ],
        out_specs=[
            pl.BlockSpec((MB, oc), lambda i: (i, 0)),
            pl.BlockSpec((1, oc), lambda i: (0, 0)),
            pl.BlockSpec((1, oc), lambda i: (0, 0)),
        ],
        out_shape=[
            jax.ShapeDtypeStruct((N, oc), jnp.float32),
            jax.ShapeDtypeStruct((1, oc), jnp.float32),
            jax.ShapeDtypeStruct((1, oc), jnp.float32),
        ],
    )(h, w, s, dis8, b)


def _comb2_body(h, w0, w2, sa, sc, d8, b, pre, sm, sq):
    d = _dis_of(d8[...])
    p = (jnp.dot(h[...], w0[...] - w2[...], preferred_element_type=jnp.float32)
         - d * sa[...] + 2.0 * d * sc[...] + b[...])
    pre[...] = p
    _stats_update(pl.program_id(0), p, sm, sq)


def _comb2(h, w0, w2, sa, sc, dis8, b):
    k, oc = w0.shape
    return pl.pallas_call(
        _comb2_body,
        grid=(_GRID,),
        in_specs=[
            pl.BlockSpec((MB, k), lambda i: (i, 0)),
            pl.BlockSpec((k, oc), lambda i: (0, 0)),
            pl.BlockSpec((k, oc), lambda i: (0, 0)),
            pl.BlockSpec((MB, oc), lambda i: (i, 0)),
            pl.BlockSpec((MB, oc), lambda i: (i, 0)),
            pl.BlockSpec((MB, 8), lambda i: (i, 0)),
            pl.BlockSpec((1, oc), lambda i: (0, 0)),
        ],
        out_specs=[
            pl.BlockSpec((MB, oc), lambda i: (i, 0)),
            pl.BlockSpec((1, oc), lambda i: (0, 0)),
            pl.BlockSpec((1, oc), lambda i: (0, 0)),
        ],
        out_shape=[
            jax.ShapeDtypeStruct((N, oc), jnp.float32),
            jax.ShapeDtypeStruct((1, oc), jnp.float32),
            jax.ShapeDtypeStruct((1, oc), jnp.float32),
        ],
    )(h, w0, w2, sa, sc, dis8, b)


def _mm3_body(h, v1, s2, w0, w1, w2, d8, b, pre, sm, sq):
    d = _dis_of(d8[...])
    p = (jnp.dot(h[...], w0[...] - w2[...], preferred_element_type=jnp.float32)
         - jnp.dot(v1[...], w1[...], preferred_element_type=jnp.float32)
         - 2.0 * jnp.dot(d * s2[...], w2[...],
                         preferred_element_type=jnp.float32)
         + b[...])
    pre[...] = p
    _stats_update(pl.program_id(0), p, sm, sq)


def _mm3(h, v1, s2, w0, w1, w2, dis8, b):
    k, oc = w0.shape
    return pl.pallas_call(
        _mm3_body,
        grid=(_GRID,),
        in_specs=[
            pl.BlockSpec((MB, k), lambda i: (i, 0)),
            pl.BlockSpec((MB, k), lambda i: (i, 0)),
            pl.BlockSpec((MB, k), lambda i: (i, 0)),
            pl.BlockSpec((k, oc), lambda i: (0, 0)),
            pl.BlockSpec((k, oc), lambda i: (0, 0)),
            pl.BlockSpec((k, oc), lambda i: (0, 0)),
            pl.BlockSpec((MB, 8), lambda i: (i, 0)),
            pl.BlockSpec((1, oc), lambda i: (0, 0)),
        ],
        out_specs=[
            pl.BlockSpec((MB, oc), lambda i: (i, 0)),
            pl.BlockSpec((1, oc), lambda i: (0, 0)),
            pl.BlockSpec((1, oc), lambda i: (0, 0)),
        ],
        out_shape=[
            jax.ShapeDtypeStruct((N, oc), jnp.float32),
            jax.ShapeDtypeStruct((1, oc), jnp.float32),
            jax.ShapeDtypeStruct((1, oc), jnp.float32),
        ],
    )(h, v1, s2, w0, w1, w2, dis8, b)


def _normact_body(want_u, pre, sm, sq, bw, bb, bms, d8, h, *u):
    mean = sm[...] * (1.0 / N)
    m2 = mean * bms[...]
    var = sq[...] * (1.0 / N) - 2.0 * m2 * mean + m2 * m2
    y = bw[...] * (pre[...] - m2) * lax.rsqrt(var + 1e-5) + bb[...]
    hv = jnp.where(y > 0.0, y, 0.2 * y)
    h[...] = hv
    if want_u:
        u[0][...] = _dis_of(d8[...]) * hv


def _normact(pre, sm, sq, bn, dis8, want_u):
    oc = pre.shape[1]
    outs = [jax.ShapeDtypeStruct((N, oc), jnp.float32)]
    ospecs = [pl.BlockSpec((MB, oc), lambda i: (i, 0))]
    if want_u:
        outs.append(jax.ShapeDtypeStruct((N, oc), jnp.float32))
        ospecs.append(pl.BlockSpec((MB, oc), lambda i: (i, 0)))
    res = pl.pallas_call(
        functools.partial(_normact_body, want_u),
        grid=(_GRID,),
        in_specs=[
            pl.BlockSpec((MB, oc), lambda i: (i, 0)),
            pl.BlockSpec((1, oc), lambda i: (0, 0)),
            pl.BlockSpec((1, oc), lambda i: (0, 0)),
            pl.BlockSpec((1, oc), lambda i: (0, 0)),
            pl.BlockSpec((1, oc), lambda i: (0, 0)),
            pl.BlockSpec((1, oc), lambda i: (0, 0)),
            pl.BlockSpec((MB, 8), lambda i: (i, 0)),
        ],
        out_specs=ospecs if want_u else ospecs[0],
        out_shape=outs if want_u else outs[0],
    )(pre, sm, sq, bn['weight'], bn['bias'], bn['mean_scale'], dis8)
    return res


def _uv_body(s, d8, v1, u1):
    d = _dis_of(d8[...])
    v = d * s[...]
    v1[...] = v
    u1[...] = -d * v


def _uv(s, dis8):
    oc = s.shape[1]
    return pl.pallas_call(
        _uv_body,
        grid=(_GRID,),
        in_specs=[
            pl.BlockSpec((MB, oc), lambda i: (i, 0)),
            pl.BlockSpec((MB, 8), lambda i: (i, 0)),
        ],
        out_specs=[
            pl.BlockSpec((MB, oc), lambda i: (i, 0)),
            pl.BlockSpec((MB, oc), lambda i: (i, 0)),
        ],
        out_shape=[
            jax.ShapeDtypeStruct((N, oc), jnp.float32),
            jax.ShapeDtypeStruct((N, oc), jnp.float32),
        ],
    )(s, dis8)


def _ucof_body(s, d8, u):
    d = _dis_of(d8[...])
    u[...] = d * d * s[...]


def _ucof(s, dis8):
    oc = s.shape[1]
    return pl.pallas_call(
        _ucof_body,
        grid=(_GRID,),
        in_specs=[
            pl.BlockSpec((MB, oc), lambda i: (i, 0)),
            pl.BlockSpec((MB, 8), lambda i: (i, 0)),
        ],
        out_specs=pl.BlockSpec((MB, oc), lambda i: (i, 0)),
        out_shape=jax.ShapeDtypeStruct((N, oc), jnp.float32),
    )(s, dis8)


def _res_body(h3, x, s, w0, w1, d8, b, out):
    d = _dis_of(d8[...])
    p = (jnp.dot(h3[...], w0[...], preferred_element_type=jnp.float32)
         - jnp.dot(d * s[...], w1[...], preferred_element_type=jnp.float32)
         + b[...] + x[...])
    out[...] = jnp.maximum(p, 0.0)


def _resconv(h3, x, s, w0, w1, dis8, b):
    k, oc = w0.shape
    return pl.pallas_call(
        _res_body,
        grid=(_GRID,),
        in_specs=[
            pl.BlockSpec((MB, k), lambda i: (i, 0)),
            pl.BlockSpec((MB, oc), lambda i: (i, 0)),
            pl.BlockSpec((MB, k), lambda i: (i, 0)),
            pl.BlockSpec((k, oc), lambda i: (0, 0)),
            pl.BlockSpec((k, oc), lambda i: (0, 0)),
            pl.BlockSpec((MB, 8), lambda i: (i, 0)),
            pl.BlockSpec((1, oc), lambda i: (0, 0)),
        ],
        out_specs=pl.BlockSpec((MB, oc), lambda i: (i, 0)),
        out_shape=jax.ShapeDtypeStruct((N, oc), jnp.float32),
    )(h3, x, s, w0, w1, dis8, b)


def _head_body(h, b3, w1, b1, w2, b2, out, pacc, cacc):
    i = pl.program_id(0)

    @pl.when(i == 0)
    def _():
        pacc[...] = jnp.zeros_like(pacc)
        cacc[...] = jnp.zeros_like(cacc)

    bids = b3[0]
    onehot = (lax.broadcasted_iota(jnp.int32, (G, MB), 0)
              == bids).astype(jnp.float32)
    pacc[...] += jnp.dot(onehot, h[...], preferred_element_type=jnp.float32)
    cacc[...] += jnp.broadcast_to(
        jnp.sum(onehot, axis=1, keepdims=True), (G, 8))

    @pl.when(i == _GRID - 1)
    def _():
        pooled = pacc[...] / jnp.maximum(cacc[:, :1], 1.0)
        o = jnp.tanh(jnp.dot(pooled, w1[...],
                             preferred_element_type=jnp.float32) + b1[...])
        out[...] = jnp.dot(o, w2[...],
                           preferred_element_type=jnp.float32) + b2[...]


def _head(h, batch3, w1, b1, w2, b2):
    return pl.pallas_call(
        _head_body,
        grid=(_GRID,),
        in_specs=[
            pl.BlockSpec((MB, 512), lambda i: (i, 0)),
            pl.BlockSpec((1, 1, MB), lambda i: (i, 0, 0)),
            pl.BlockSpec((512, 64), lambda i: (0, 0)),
            pl.BlockSpec((1, 64), lambda i: (0, 0)),
            pl.BlockSpec((64, 12), lambda i: (0, 0)),
            pl.BlockSpec((1, 12), lambda i: (0, 0)),
        ],
        out_specs=pl.BlockSpec((G, 12), lambda i: (0, 0)),
        out_shape=jax.ShapeDtypeStruct((G, 12), jnp.float32),
        scratch_shapes=[
            pltpu.VMEM((G, 512), jnp.float32),
            pltpu.VMEM((G, 8), jnp.float32),
        ],
    )(h, batch3, w1, b1, w2, b2)


def kernel(x, edge_index, edge_weight, batch, params):
    del edge_weight
    src = edge_index[0]
    dst = edge_index[1]

    def windows(scatter_ids, gather_ids):
        perm = jnp.argsort((scatter_ids >= NH).astype(jnp.int32))
        sp = scatter_ids[perm]
        gp = gather_ids[perm]
        cnt0 = jnp.sum((scatter_ids < NH).astype(jnp.int32))
        t = jnp.arange(NT, dtype=jnp.int32)
        bounds = jnp.stack([
            (cnt0 * t) // NT, (cnt0 * (t + 1)) // NT,
            cnt0 + ((E - cnt0) * t) // NT,
            cnt0 + ((E - cnt0) * (t + 1)) // NT,
        ]).reshape(2, 2, NT)
        start = bounds[:, 0]
        end = bounds[:, 1]
        astart = jnp.minimum(start - (start % 128), E - WIN)
        npairs = (end - astart + 2 * CH - 1) // (2 * CH)
        meta = jnp.stack(
            [astart, start, end, npairs] + [jnp.zeros_like(start)] * 12,
            axis=-1).astype(jnp.int32).reshape(2, NT, 1, 16)
        gidx = jnp.stack([2 * gp, 2 * gp + 1]).reshape(2, 1, E)
        return gidx, sp, meta

    gidx, dstp, meta = windows(dst, src)
    gidx_d, srcp_d, meta_d = windows(src, src)

    ones_t = jnp.ones((2 * N, FH), jnp.float32)
    degf = _prop_call(ones_t, gidx_d, srcp_d, meta_d)
    dis8 = _disprep(degf)

    def t2(a):
        return a.reshape(2 * N, FH)

    def b2d(v):
        return v.reshape(1, -1)

    h = x
    for blk in range(4):
        p1 = params['conv%d' % (blk * 4 + 1)]
        p2 = params['conv%d' % (blk * 4 + 2)]
        p3 = params['conv%d' % (blk * 4 + 3)]
        p4 = params['conv%d' % (blk * 4 + 4)]
        bn1 = params['bn%d' % (blk * 4 + 1)]
        bn2 = params['bn%d' % (blk * 4 + 2)]
        bn3 = params['bn%d' % (blk * 4 + 3)]
        bn1 = {k: b2d(v) for k, v in bn1.items()}
        bn2 = {k: b2d(v) for k, v in bn2.items()}
        bn3 = {k: b2d(v) for k, v in bn3.items()}

        y1u = _mm_scale(h, p1['Ws'][1], dis8)
        s1 = _prop(t2(y1u), gidx, dstp, meta)
        pre1, sm, sq = _comb1(h, p1['Ws'][0], s1, dis8, b2d(p1['b']))
        h1, u0 = _normact(pre1, sm, sq, bn1, dis8, True)

        s1b = _prop(t2(u0), gidx, dstp, meta)
        v1, u1 = _uv(s1b, dis8)
        s2 = _prop(t2(u1), gidx, dstp, meta)
        pre2, sm, sq = _mm3(h1, v1, s2, p2['Ws'][0], p2['Ws'][1],
                            p2['Ws'][2], dis8, b2d(p2['b']))
        h2 = _normact(pre2, sm, sq, bn2, dis8, False)

        y1u = _mm_scale(h2, p3['Ws'][1], dis8)
        y2u = _mm_scale(h2, p3['Ws'][2], dis8)
        sa = _prop(t2(y1u), gidx, dstp, meta)
        sb = _prop(t2(y2u), gidx, dstp, meta)
        uc = _ucof(sb, dis8)
        sc = _prop(t2(uc), gidx, dstp, meta)
        pre3, sm, sq = _comb2(h2, p3['Ws'][0], p3['Ws'][2], sa, sc,
                              dis8, b2d(p3['b']))
        h3, u3 = _normact(pre3, sm, sq, bn3, dis8, True)

        s4 = _prop(t2(u3), gidx, dstp, meta)
        h = _resconv(h3, x, s4, p4['Ws'][0], p4['Ws'][1], dis8, b2d(p4['b']))

    batch3 = batch.reshape(_GRID, 1, MB)
    return _head(h, batch3, params['linear1']['W'], b2d(params['linear1']['b']),
                 params['linear2']['W'], b2d(params['linear2']['b']))
